# Initial kernel scaffold; baseline (speedup 1.0000x reference)
import jax
import jax.numpy as jnp
from jax.experimental import pallas as pl


def kernel(xyz0, xyz1, xyz2, W00, b00, W01, b01, W10, b10, W11, b11, Wn1, bn1, Wl1, bl1, W20, b20, W21, b21, Wn2, bn2, Wl2, bl2):
    raise NotImplementedError("write your pallas kernel here")


# speedup vs baseline: 3.8290x; 3.8290x over previous
import functools

import jax
import jax.numpy as jnp
from jax.experimental import pallas as pl
from jax.experimental.pallas import tpu as pltpu

K = 16


def _lrelu(x):
    return jnp.where(x >= 0, x, 0.1 * x)


def _mlp_body(x_ref, w00, b00, w01, b01, w10, b10, w11, b11, f0_ref, g1_ref):
    x = x_ref[...]
    h = _lrelu(jnp.dot(w00[...], x, preferred_element_type=jnp.float32) + b00[...][:, None])
    f0 = _lrelu(jnp.dot(w01[...], h, preferred_element_type=jnp.float32) + b01[...][:, None])
    h2 = _lrelu(jnp.dot(w10[...], f0, preferred_element_type=jnp.float32) + b10[...][:, None])
    g1 = _lrelu(jnp.dot(w11[...], h2, preferred_element_type=jnp.float32) + b11[...][:, None])
    f0_ref[...] = f0
    g1_ref[...] = g1


def _mlp_stack(xyz, W00, b00, W01, b01, W10, b10, W11, b11):
    N = xyz.shape[1]
    BN = 2048
    grid = (N // BN,)
    f0, g1 = pl.pallas_call(
        _mlp_body,
        grid=grid,
        in_specs=[
            pl.BlockSpec((3, BN), lambda i: (0, i)),
            pl.BlockSpec((16, 3), lambda i: (0, 0)),
            pl.BlockSpec((16,), lambda i: (0,)),
            pl.BlockSpec((16, 16), lambda i: (0, 0)),
            pl.BlockSpec((16,), lambda i: (0,)),
            pl.BlockSpec((16, 16), lambda i: (0, 0)),
            pl.BlockSpec((16,), lambda i: (0,)),
            pl.BlockSpec((32, 16), lambda i: (0, 0)),
            pl.BlockSpec((32,), lambda i: (0,)),
        ],
        out_specs=[
            pl.BlockSpec((16, BN), lambda i: (0, i)),
            pl.BlockSpec((32, BN), lambda i: (0, i)),
        ],
        out_shape=[
            jax.ShapeDtypeStruct((16, N), jnp.float32),
            jax.ShapeDtypeStruct((32, N), jnp.float32),
        ],
    )(xyz, W00, b00, W01, b01, W10, b10, W11, b11)
    return f0, g1


def _knn_idx(xyz, sampled, k):
    sq_x = jnp.sum(xyz ** 2, axis=0)
    sq_s = jnp.sum(sampled ** 2, axis=0)
    inner = sampled.T @ xyz
    dist = sq_s[:, None] + sq_x[None, :] - 2.0 * inner
    _, idx = jax.lax.top_k(-dist, k)
    return idx


def _point_conv(xyz, feat, sampled, Wn, bn, Wl, bl):
    idx = _knn_idx(xyz, sampled, K)
    knn_xyz = xyz[:, idx]
    knn_off = knn_xyz - sampled[:, :, None]
    knn_feat = feat[:, idx]
    w = _lrelu(jnp.einsum('oc,cmk->omk', Wn, knn_off) + bn[:, None, None])
    out = jnp.einsum('omk,cmk->moc', w, knn_feat)
    M = out.shape[0]
    out = out.reshape(M, -1)
    out = _lrelu(out @ Wl.T + bl)
    return out.T


def _mlp_plain(x, layers):
    for W, b in layers:
        x = _lrelu(W @ x + b[:, None])
    return x


def kernel(xyz0, xyz1, xyz2, W00, b00, W01, b01, W10, b10, W11, b11,
           Wn1, bn1, Wl1, bl1, W20, b20, W21, b21, Wn2, bn2, Wl2, bl2):
    x0 = xyz0[0]
    x1 = xyz1[0]
    x2 = xyz2[0]
    f0, g1 = _mlp_stack(x0, W00, b00, W01, b01, W10, b10, W11, b11)
    f1 = _point_conv(x0, g1, x1, Wn1, bn1, Wl1, bl1)
    g2 = _mlp_plain(f1, [(W20, b20), (W21, b21)])
    f2 = _point_conv(x1, g2, x2, Wn2, bn2, Wl2, bl2)
    return (f0[None], f1[None], f2[None])
